# baseline (speedup 1.0000x reference)
import functools

import jax
import jax.numpy as jnp
from jax import lax
from jax.experimental import pallas as pl
from jax.experimental.pallas import tpu as pltpu
from jax.experimental.pallas import tpu_sc as plsc

N = 10000
D = 128
G = 64
C = 10

NC = 2
NS = 16
NW = NC * NS
CHUNK = 128
N_ACC = 10240
ROWS_PT = N_ACC // NS
NBUF = 2
NPHASE = 2
ACHUNK = 128
BR = 1000


def _sc_mesh():
    return plsc.VectorSubcoreMesh(core_axis_name="c", subcore_axis_name="s")


def _make_deg(cpt):
    @functools.partial(
        pl.kernel,
        mesh=_sc_mesh(),
        out_type=jax.ShapeDtypeStruct((NC, 1, N_ACC), jnp.float32),
        scratch_types=[
            pltpu.VMEM((cpt, CHUNK), jnp.int32),
            pltpu.VMEM((CHUNK,), jnp.float32),
            pltpu.VMEM((ROWS_PT,), jnp.float32),
            pltpu.VMEM_SHARED((N_ACC,), jnp.float32),
        ],
    )
    def deg_kernel(dsti_hbm, deg_out, dst_v, ones_v, stage_v, acc_sh):
        ci = lax.axis_index("c")
        si = lax.axis_index("s")
        w = si * NC + ci
        zf = jnp.zeros((16,), jnp.float32)
        of = jnp.ones((16,), jnp.float32)

        def zbody(i, carry):
            stage_v[pl.ds(i * 16, 16)] = zf
            return carry

        lax.fori_loop(0, ROWS_PT // 16, zbody, 0)
        for i in range(CHUNK // 16):
            ones_v[pl.ds(i * 16, 16)] = of
        pltpu.sync_copy(stage_v, acc_sh.at[pl.ds(si * ROWS_PT, ROWS_PT)])
        plsc.subcore_barrier()

        pltpu.sync_copy(dsti_hbm.at[w], dst_v)

        def body(cn, carry):
            pltpu.sync_copy(ones_v, acc_sh.at[dst_v.at[cn]], add=True)
            return carry

        lax.fori_loop(0, cpt, body, 0)
        plsc.subcore_barrier()
        pltpu.sync_copy(acc_sh.at[pl.ds(si * ROWS_PT, ROWS_PT)],
                        deg_out.at[ci, 0, pl.ds(si * ROWS_PT, ROWS_PT)])

    return deg_kernel


def _make_agg(cpt):
    @functools.partial(
        pl.kernel,
        mesh=_sc_mesh(),
        out_type=jax.ShapeDtypeStruct((NC, N_ACC, D), jnp.float32),
        scratch_types=(
            [pltpu.VMEM((cpt // NPHASE, ACHUNK), jnp.int32),
             pltpu.VMEM((cpt // NPHASE, ACHUNK), jnp.int32)]
            + [pltpu.VMEM((ACHUNK, D), jnp.float32)] * NBUF
            + [pltpu.VMEM_SHARED((N_ACC, D), jnp.float32)]
            + [pltpu.SemaphoreType.DMA] * (2 * NBUF)
        ),
    )
    def agg_kernel(h_hbm, srci_hbm, dsti_hbm, agg_out, src_v, dst_v, *rest):
        bufs = rest[:NBUF]
        acc_sh = rest[NBUF]
        semg = rest[NBUF + 1:2 * NBUF + 1]
        sems = rest[2 * NBUF + 1:]
        ci = lax.axis_index("c")
        si = lax.axis_index("s")
        w = si * NC + ci
        zf = jnp.zeros((16,), jnp.float32)

        def zbody(i, carry):
            for l in range(D // 16):
                bufs[0][i, pl.ds(l * 16, 16)] = zf
            return carry

        lax.fori_loop(0, ACHUNK, zbody, 0)
        for j in range(ROWS_PT // ACHUNK):
            pltpu.sync_copy(
                bufs[0], acc_sh.at[pl.ds(si * ROWS_PT + j * ACHUNK, ACHUNK)])
        plsc.subcore_barrier()

        def scat_wait(b):
            pltpu.make_async_copy(
                bufs[b], acc_sh.at[pl.ds(0, ACHUNK)], sems[b]).wait()

        cpp = cpt // NPHASE
        for ph in range(NPHASE):
            pltpu.sync_copy(srci_hbm.at[w, pl.ds(ph * cpp, cpp)], src_v)
            pltpu.sync_copy(dsti_hbm.at[w, pl.ds(ph * cpp, cpp)], dst_v)
            for b in range(NBUF - 1):
                pltpu.async_copy(h_hbm.at[src_v.at[b]], bufs[b], semg[b])

            def outer(g, carry):
                for b in range(NBUF):
                    cn = g * NBUF + b
                    bp = (b - 1) % NBUF
                    pltpu.make_async_copy(
                        h_hbm.at[src_v.at[cn]], bufs[b], semg[b]).wait()
                    pltpu.async_copy(
                        bufs[b], acc_sh.at[dst_v.at[cn]], sems[b], add=True)

                    @pl.when(cn > 0)
                    def _drain_prev():
                        scat_wait(bp)

                    nxt = cn + NBUF - 1

                    @pl.when(nxt < cpp)
                    def _start_next():
                        pltpu.async_copy(
                            h_hbm.at[src_v.at[nxt]], bufs[bp], semg[bp])
                return carry

            lax.fori_loop(0, cpp // NBUF, outer, 0)
            scat_wait(NBUF - 1)
        plsc.subcore_barrier()
        for j in range(ROWS_PT // ACHUNK):
            off = si * ROWS_PT + j * ACHUNK
            pltpu.sync_copy(acc_sh.at[pl.ds(off, ACHUNK)], bufs[0])
            pltpu.sync_copy(bufs[0], agg_out.at[ci, pl.ds(off, ACHUNK)])

    return agg_kernel


def _mm1_body(x_ref, w_ref, deg_ref, o_ref):
    dinv = lax.rsqrt(deg_ref[...])
    h = jnp.dot(x_ref[...], w_ref[...], preferred_element_type=jnp.float32)
    o_ref[...] = h * dinv


def _mm1(x, W1, deg_col):
    return pl.pallas_call(
        _mm1_body,
        grid=(N // BR,),
        in_specs=[
            pl.BlockSpec((BR, D), lambda i: (i, 0)),
            pl.BlockSpec((D, D), lambda i: (0, 0)),
            pl.BlockSpec((BR, 1), lambda i: (i, 0)),
        ],
        out_specs=pl.BlockSpec((BR, D), lambda i: (i, 0)),
        out_shape=jax.ShapeDtypeStruct((N, D), jnp.float32),
    )(x, W1, deg_col)


def _mid_body(agg_ref, h_ref, deg_ref, b_ref, w_ref, o_ref):
    dinv = lax.rsqrt(deg_ref[...])
    t = (agg_ref[0] + agg_ref[1] + h_ref[...]) * dinv + b_ref[...]
    r = jnp.maximum(t, 0.0)
    o_ref[...] = jnp.dot(r, w_ref[...],
                         preferred_element_type=jnp.float32) * dinv


def _mid(agg, h1p, deg_col, b1, W2):
    return pl.pallas_call(
        _mid_body,
        grid=(N // BR,),
        in_specs=[
            pl.BlockSpec((NC, BR, D), lambda i: (0, i, 0)),
            pl.BlockSpec((BR, D), lambda i: (i, 0)),
            pl.BlockSpec((BR, 1), lambda i: (i, 0)),
            pl.BlockSpec((1, D), lambda i: (0, 0)),
            pl.BlockSpec((D, D), lambda i: (0, 0)),
        ],
        out_specs=pl.BlockSpec((BR, D), lambda i: (i, 0)),
        out_shape=jax.ShapeDtypeStruct((N, D), jnp.float32),
    )(agg, h1p, deg_col, b1, W2)


def _final_body(agg_ref, h_ref, deg_ref, b_ref, batch_ref, wout_ref, bout_ref,
                o_ref, pooled):
    i = pl.program_id(0)
    dinv = lax.rsqrt(deg_ref[...])
    t = (agg_ref[0] + agg_ref[1] + h_ref[...]) * dinv + b_ref[...]
    gids = lax.broadcasted_iota(jnp.int32, (G, BR), 0)
    onehot = jnp.where(batch_ref[0] == gids, 1.0, 0.0)
    part = jnp.dot(onehot, t, preferred_element_type=jnp.float32)

    @pl.when(i == 0)
    def _init():
        pooled[...] = part

    @pl.when(i > 0)
    def _acc():
        pooled[...] += part

    @pl.when(i == pl.num_programs(0) - 1)
    def _fin():
        logits = jnp.dot(pooled[...], wout_ref[...],
                         preferred_element_type=jnp.float32) + bout_ref[...]
        m = jnp.max(logits, axis=-1, keepdims=True)
        s = logits - m
        lse = jnp.log(jnp.sum(jnp.exp(s), axis=-1, keepdims=True))
        o_ref[...] = s - lse


def _final(agg, h2p, deg_col, b2, batch2d, W_out, b_out):
    return pl.pallas_call(
        _final_body,
        grid=(N // BR,),
        in_specs=[
            pl.BlockSpec((NC, BR, D), lambda i: (0, i, 0)),
            pl.BlockSpec((BR, D), lambda i: (i, 0)),
            pl.BlockSpec((BR, 1), lambda i: (i, 0)),
            pl.BlockSpec((1, D), lambda i: (0, 0)),
            pl.BlockSpec((1, 1, BR), lambda i: (i, 0, 0)),
            pl.BlockSpec((D, C), lambda i: (0, 0)),
            pl.BlockSpec((1, C), lambda i: (0, 0)),
        ],
        out_specs=pl.BlockSpec((G, C), lambda i: (0, 0)),
        out_shape=jax.ShapeDtypeStruct((G, C), jnp.float32),
        scratch_shapes=[pltpu.VMEM((G, D), jnp.float32)],
    )(agg, h2p, deg_col, b2, batch2d, W_out, b_out)


def kernel(x, edge_index, batch, W1, b1, W2, b2, W_out, b_out):
    src = edge_index[0]
    dst = edge_index[1]
    e = src.shape[0]

    def pad_to(idx, fill_mod, fill_base, chunk, cpt):
        pad = NW * chunk * cpt - e
        ar = jnp.arange(pad, dtype=jnp.int32)
        return jnp.concatenate([idx, fill_base + ar % fill_mod]).reshape(
            NW, cpt, chunk)

    cpt_d = -(-e // (NW * CHUNK))
    dst_pd = pad_to(dst, N_ACC - N, N, CHUNK, cpt_d)
    cpt = -(-e // (NW * ACHUNK))
    cpt += (-cpt) % (NBUF * NPHASE * 2)
    src_p = pad_to(src, N, 0, ACHUNK, cpt)
    dst_p = pad_to(dst, N_ACC - N, N, ACHUNK, cpt)

    deg2 = _make_deg(cpt_d)(dst_pd)
    deg_col = (deg2[0, 0, :N] + deg2[1, 0, :N] + 1.0).reshape(N, 1)

    h1p = _mm1(x, W1, deg_col)
    agg1 = _make_agg(cpt)(h1p, src_p, dst_p)
    h2p = _mid(agg1, h1p, deg_col, b1.reshape(1, D), W2)
    agg2 = _make_agg(cpt)(h2p, src_p, dst_p)
    batch3d = batch.reshape(N // BR, 1, BR)
    return _final(agg2, h2p, deg_col, b2.reshape(1, D), batch3d,
                  W_out, b_out.reshape(1, C))


# speedup vs baseline: 1.1470x; 1.1470x over previous
import functools

import jax
import jax.numpy as jnp
from jax import lax
from jax.experimental import pallas as pl
from jax.experimental.pallas import tpu as pltpu
from jax.experimental.pallas import tpu_sc as plsc

N = 10000
D = 128
G = 64
C = 10

NC = 2
NS = 16
NW = NC * NS
CHUNK = 128
N_ACC = 10240
ROWS_PT = N_ACC // NS
NBUF = 2
NPHASE = 2
ACHUNK = 128
BR = 1000


def _sc_mesh():
    return plsc.VectorSubcoreMesh(core_axis_name="c", subcore_axis_name="s")


def _make_deg(cpt):
    @functools.partial(
        pl.kernel,
        mesh=_sc_mesh(),
        out_type=jax.ShapeDtypeStruct((NC, 1, N_ACC), jnp.float32),
        scratch_types=[
            pltpu.VMEM((cpt, CHUNK), jnp.int32),
            pltpu.VMEM((CHUNK,), jnp.float32),
            pltpu.VMEM((ROWS_PT,), jnp.float32),
            pltpu.VMEM_SHARED((N_ACC,), jnp.float32),
        ],
    )
    def deg_kernel(dsti_hbm, deg_out, dst_v, ones_v, stage_v, acc_sh):
        ci = lax.axis_index("c")
        si = lax.axis_index("s")
        w = si * NC + ci
        zf = jnp.zeros((16,), jnp.float32)
        of = jnp.ones((16,), jnp.float32)

        def zbody(i, carry):
            stage_v[pl.ds(i * 16, 16)] = zf
            return carry

        lax.fori_loop(0, ROWS_PT // 16, zbody, 0)
        for i in range(CHUNK // 16):
            ones_v[pl.ds(i * 16, 16)] = of
        pltpu.sync_copy(stage_v, acc_sh.at[pl.ds(si * ROWS_PT, ROWS_PT)])
        plsc.subcore_barrier()

        pltpu.sync_copy(dsti_hbm.at[w], dst_v)

        def body(cn, carry):
            pltpu.sync_copy(ones_v, acc_sh.at[dst_v.at[cn]], add=True)
            return carry

        lax.fori_loop(0, cpt, body, 0)
        plsc.subcore_barrier()
        pltpu.sync_copy(acc_sh.at[pl.ds(si * ROWS_PT, ROWS_PT)],
                        deg_out.at[ci, 0, pl.ds(si * ROWS_PT, ROWS_PT)])

    return deg_kernel


def _make_agg(cpt):
    @functools.partial(
        pl.kernel,
        mesh=_sc_mesh(),
        out_type=jax.ShapeDtypeStruct((NC, N_ACC, D), jnp.float32),
        scratch_types=(
            [pltpu.VMEM((cpt // NPHASE, ACHUNK), jnp.int32),
             pltpu.VMEM((cpt // NPHASE, ACHUNK), jnp.int32)]
            + [pltpu.VMEM((ACHUNK, D), jnp.float32)] * NBUF
            + [pltpu.VMEM_SHARED((N_ACC, D), jnp.float32)]
            + [pltpu.SemaphoreType.DMA] * (2 * NBUF)
        ),
    )
    def agg_kernel(h_hbm, srci_hbm, dsti_hbm, agg_out, src_v, dst_v, *rest):
        bufs = rest[:NBUF]
        acc_sh = rest[NBUF]
        semg = rest[NBUF + 1:2 * NBUF + 1]
        sems = rest[2 * NBUF + 1:]
        ci = lax.axis_index("c")
        si = lax.axis_index("s")
        w = si * NC + ci
        zf = jnp.zeros((16,), jnp.float32)

        def zbody(i, carry):
            for l in range(D // 16):
                bufs[0][i, pl.ds(l * 16, 16)] = zf
            return carry

        lax.fori_loop(0, ACHUNK, zbody, 0)
        for j in range(ROWS_PT // ACHUNK):
            pltpu.sync_copy(
                bufs[0], acc_sh.at[pl.ds(si * ROWS_PT + j * ACHUNK, ACHUNK)])
        plsc.subcore_barrier()

        cpp = cpt // NPHASE
        for ph in range(NPHASE):
            pltpu.sync_copy(srci_hbm.at[w, pl.ds(ph * cpp, cpp)], src_v)
            pltpu.sync_copy(dsti_hbm.at[w, pl.ds(ph * cpp, cpp)], dst_v)
            for b in range(NBUF):
                pltpu.async_copy(h_hbm.at[src_v.at[b]], bufs[b], semg[b])

            def outer(g, carry):
                for b in range(NBUF):
                    cn = g * NBUF + b
                    pltpu.make_async_copy(
                        h_hbm.at[src_v.at[cn]], bufs[b], semg[b]).wait()
                    pltpu.sync_copy(bufs[b], acc_sh.at[dst_v.at[cn]], add=True)
                    nxt = cn + NBUF

                    @pl.when(nxt < cpp)
                    def _start_next():
                        pltpu.async_copy(
                            h_hbm.at[src_v.at[nxt]], bufs[b], semg[b])
                return carry

            lax.fori_loop(0, cpp // NBUF, outer, 0)
        plsc.subcore_barrier()
        for j in range(ROWS_PT // ACHUNK):
            off = si * ROWS_PT + j * ACHUNK
            pltpu.sync_copy(acc_sh.at[pl.ds(off, ACHUNK)], bufs[0])
            pltpu.sync_copy(bufs[0], agg_out.at[ci, pl.ds(off, ACHUNK)])

    return agg_kernel


def _mm1_body(x_ref, w_ref, deg_ref, o_ref):
    dinv = lax.rsqrt(deg_ref[...])
    h = jnp.dot(x_ref[...], w_ref[...], preferred_element_type=jnp.float32)
    o_ref[...] = h * dinv


def _mm1(x, W1, deg_col):
    return pl.pallas_call(
        _mm1_body,
        grid=(N // BR,),
        in_specs=[
            pl.BlockSpec((BR, D), lambda i: (i, 0)),
            pl.BlockSpec((D, D), lambda i: (0, 0)),
            pl.BlockSpec((BR, 1), lambda i: (i, 0)),
        ],
        out_specs=pl.BlockSpec((BR, D), lambda i: (i, 0)),
        out_shape=jax.ShapeDtypeStruct((N, D), jnp.float32),
    )(x, W1, deg_col)


def _mid_body(agg_ref, h_ref, deg_ref, b_ref, w_ref, o_ref):
    dinv = lax.rsqrt(deg_ref[...])
    t = (agg_ref[0] + agg_ref[1] + h_ref[...]) * dinv + b_ref[...]
    r = jnp.maximum(t, 0.0)
    o_ref[...] = jnp.dot(r, w_ref[...],
                         preferred_element_type=jnp.float32) * dinv


def _mid(agg, h1p, deg_col, b1, W2):
    return pl.pallas_call(
        _mid_body,
        grid=(N // BR,),
        in_specs=[
            pl.BlockSpec((NC, BR, D), lambda i: (0, i, 0)),
            pl.BlockSpec((BR, D), lambda i: (i, 0)),
            pl.BlockSpec((BR, 1), lambda i: (i, 0)),
            pl.BlockSpec((1, D), lambda i: (0, 0)),
            pl.BlockSpec((D, D), lambda i: (0, 0)),
        ],
        out_specs=pl.BlockSpec((BR, D), lambda i: (i, 0)),
        out_shape=jax.ShapeDtypeStruct((N, D), jnp.float32),
    )(agg, h1p, deg_col, b1, W2)


def _final_body(agg_ref, h_ref, deg_ref, b_ref, batch_ref, wout_ref, bout_ref,
                o_ref, pooled):
    i = pl.program_id(0)
    dinv = lax.rsqrt(deg_ref[...])
    t = (agg_ref[0] + agg_ref[1] + h_ref[...]) * dinv + b_ref[...]
    gids = lax.broadcasted_iota(jnp.int32, (G, BR), 0)
    onehot = jnp.where(batch_ref[0] == gids, 1.0, 0.0)
    part = jnp.dot(onehot, t, preferred_element_type=jnp.float32)

    @pl.when(i == 0)
    def _init():
        pooled[...] = part

    @pl.when(i > 0)
    def _acc():
        pooled[...] += part

    @pl.when(i == pl.num_programs(0) - 1)
    def _fin():
        logits = jnp.dot(pooled[...], wout_ref[...],
                         preferred_element_type=jnp.float32) + bout_ref[...]
        m = jnp.max(logits, axis=-1, keepdims=True)
        s = logits - m
        lse = jnp.log(jnp.sum(jnp.exp(s), axis=-1, keepdims=True))
        o_ref[...] = s - lse


def _final(agg, h2p, deg_col, b2, batch2d, W_out, b_out):
    return pl.pallas_call(
        _final_body,
        grid=(N // BR,),
        in_specs=[
            pl.BlockSpec((NC, BR, D), lambda i: (0, i, 0)),
            pl.BlockSpec((BR, D), lambda i: (i, 0)),
            pl.BlockSpec((BR, 1), lambda i: (i, 0)),
            pl.BlockSpec((1, D), lambda i: (0, 0)),
            pl.BlockSpec((1, 1, BR), lambda i: (i, 0, 0)),
            pl.BlockSpec((D, C), lambda i: (0, 0)),
            pl.BlockSpec((1, C), lambda i: (0, 0)),
        ],
        out_specs=pl.BlockSpec((G, C), lambda i: (0, 0)),
        out_shape=jax.ShapeDtypeStruct((G, C), jnp.float32),
        scratch_shapes=[pltpu.VMEM((G, D), jnp.float32)],
    )(agg, h2p, deg_col, b2, batch2d, W_out, b_out)


def kernel(x, edge_index, batch, W1, b1, W2, b2, W_out, b_out):
    src = edge_index[0]
    dst = edge_index[1]
    e = src.shape[0]

    def pad_to(idx, fill_mod, fill_base, chunk, cpt):
        pad = NW * chunk * cpt - e
        ar = jnp.arange(pad, dtype=jnp.int32)
        return jnp.concatenate([idx, fill_base + ar % fill_mod]).reshape(
            NW, cpt, chunk)

    cpt_d = -(-e // (NW * CHUNK))
    dst_pd = pad_to(dst, N_ACC - N, N, CHUNK, cpt_d)
    cpt = -(-e // (NW * ACHUNK))
    cpt += (-cpt) % (NBUF * NPHASE * 2)
    src_p = pad_to(src, N, 0, ACHUNK, cpt)
    dst_p = pad_to(dst, N_ACC - N, N, ACHUNK, cpt)

    deg2 = _make_deg(cpt_d)(dst_pd)
    deg_col = (deg2[0, 0, :N] + deg2[1, 0, :N] + 1.0).reshape(N, 1)

    h1p = _mm1(x, W1, deg_col)
    agg1 = _make_agg(cpt)(h1p, src_p, dst_p)
    h2p = _mid(agg1, h1p, deg_col, b1.reshape(1, D), W2)
    agg2 = _make_agg(cpt)(h2p, src_p, dst_p)
    batch3d = batch.reshape(N // BR, 1, BR)
    return _final(agg2, h2p, deg_col, b2.reshape(1, D), batch3d,
                  W_out, b_out.reshape(1, C))


# speedup vs baseline: 1.1919x; 1.0391x over previous
import functools

import jax
import jax.numpy as jnp
from jax import lax
from jax.experimental import pallas as pl
from jax.experimental.pallas import tpu as pltpu
from jax.experimental.pallas import tpu_sc as plsc

N = 10000
D = 128
G = 64
C = 10

NC = 2
NS = 16
NW = NC * NS
CHUNK = 128
N_ACC = 10240
ROWS_PT = N_ACC // NS
NBUF = 2
NPHASE = 2
ACHUNK = 128
BR = 1000


def _sc_mesh():
    return plsc.VectorSubcoreMesh(core_axis_name="c", subcore_axis_name="s")


def _make_deg(cpt):
    @functools.partial(
        pl.kernel,
        mesh=_sc_mesh(),
        out_type=jax.ShapeDtypeStruct((NC, 1, N_ACC), jnp.float32),
        scratch_types=[
            pltpu.VMEM((cpt, CHUNK), jnp.int32),
            pltpu.VMEM((CHUNK,), jnp.float32),
            pltpu.VMEM((ROWS_PT,), jnp.float32),
            pltpu.VMEM_SHARED((N_ACC,), jnp.float32),
        ],
    )
    def deg_kernel(ei_hbm, pad_hbm, deg_out, dst_v, ones_v, stage_v, acc_sh):
        ci = lax.axis_index("c")
        si = lax.axis_index("s")
        w = si * NC + ci
        zf = jnp.zeros((16,), jnp.float32)
        of = jnp.ones((16,), jnp.float32)

        def zbody(i, carry):
            stage_v[pl.ds(i * 16, 16)] = zf
            return carry

        lax.fori_loop(0, ROWS_PT // 16, zbody, 0)
        for i in range(CHUNK // 16):
            ones_v[pl.ds(i * 16, 16)] = of
        pltpu.sync_copy(stage_v, acc_sh.at[pl.ds(si * ROWS_PT, ROWS_PT)])
        plsc.subcore_barrier()

        @pl.when(w == NW - 1)
        def _load_tail():
            pltpu.sync_copy(pad_hbm.at[1], dst_v)

        @pl.when(w < NW - 1)
        def _load_main():
            pltpu.sync_copy(ei_hbm.at[1, pl.ds(w * cpt, cpt)], dst_v)

        def body(cn, carry):
            pltpu.sync_copy(ones_v, acc_sh.at[dst_v.at[cn]], add=True)
            return carry

        lax.fori_loop(0, cpt, body, 0)
        plsc.subcore_barrier()
        pltpu.sync_copy(acc_sh.at[pl.ds(si * ROWS_PT, ROWS_PT)],
                        deg_out.at[ci, 0, pl.ds(si * ROWS_PT, ROWS_PT)])

    return deg_kernel


def _make_agg(cpt):
    @functools.partial(
        pl.kernel,
        mesh=_sc_mesh(),
        out_type=jax.ShapeDtypeStruct((NC, N_ACC, D), jnp.float32),
        scratch_types=(
            [pltpu.VMEM((cpt // NPHASE, ACHUNK), jnp.int32),
             pltpu.VMEM((cpt // NPHASE, ACHUNK), jnp.int32)]
            + [pltpu.VMEM((ACHUNK, D), jnp.float32)] * NBUF
            + [pltpu.VMEM_SHARED((N_ACC, D), jnp.float32)]
            + [pltpu.SemaphoreType.DMA] * NBUF
        ),
    )
    def agg_kernel(h_hbm, ei_hbm, pad_hbm, agg_out, src_v, dst_v, *rest):
        bufs = rest[:NBUF]
        acc_sh = rest[NBUF]
        semg = rest[NBUF + 1:2 * NBUF + 1]
        ci = lax.axis_index("c")
        si = lax.axis_index("s")
        w = si * NC + ci
        zf = jnp.zeros((16,), jnp.float32)

        def zbody(i, carry):
            for l in range(D // 16):
                bufs[0][i, pl.ds(l * 16, 16)] = zf
            return carry

        lax.fori_loop(0, ACHUNK, zbody, 0)
        for j in range(ROWS_PT // ACHUNK):
            pltpu.sync_copy(
                bufs[0], acc_sh.at[pl.ds(si * ROWS_PT + j * ACHUNK, ACHUNK)])
        plsc.subcore_barrier()

        cpp = cpt // NPHASE
        for ph in range(NPHASE):
            @pl.when(w == NW - 1)
            def _load_tail():
                pltpu.sync_copy(pad_hbm.at[0, pl.ds(ph * cpp, cpp)], src_v)
                pltpu.sync_copy(pad_hbm.at[1, pl.ds(ph * cpp, cpp)], dst_v)

            @pl.when(w < NW - 1)
            def _load_main():
                off = w * cpt + ph * cpp
                pltpu.sync_copy(ei_hbm.at[0, pl.ds(off, cpp)], src_v)
                pltpu.sync_copy(ei_hbm.at[1, pl.ds(off, cpp)], dst_v)
            for b in range(NBUF):
                pltpu.async_copy(h_hbm.at[src_v.at[b]], bufs[b], semg[b])

            def outer(g, carry):
                for b in range(NBUF):
                    cn = g * NBUF + b
                    pltpu.make_async_copy(
                        h_hbm.at[src_v.at[cn]], bufs[b], semg[b]).wait()
                    pltpu.sync_copy(bufs[b], acc_sh.at[dst_v.at[cn]], add=True)
                    nxt = cn + NBUF

                    @pl.when(nxt < cpp)
                    def _start_next():
                        pltpu.async_copy(
                            h_hbm.at[src_v.at[nxt]], bufs[b], semg[b])
                return carry

            lax.fori_loop(0, cpp // NBUF, outer, 0)
        plsc.subcore_barrier()
        for j in range(ROWS_PT // ACHUNK):
            off = si * ROWS_PT + j * ACHUNK
            pltpu.sync_copy(acc_sh.at[pl.ds(off, ACHUNK)], bufs[0])
            pltpu.sync_copy(bufs[0], agg_out.at[ci, pl.ds(off, ACHUNK)])

    return agg_kernel


def _mm1_body(x_ref, w_ref, deg_ref, o_ref):
    dinv = lax.rsqrt(deg_ref[...])
    h = jnp.dot(x_ref[...], w_ref[...], preferred_element_type=jnp.float32)
    o_ref[...] = h * dinv


def _mm1(x, W1, deg_col):
    return pl.pallas_call(
        _mm1_body,
        grid=(N // BR,),
        in_specs=[
            pl.BlockSpec((BR, D), lambda i: (i, 0)),
            pl.BlockSpec((D, D), lambda i: (0, 0)),
            pl.BlockSpec((BR, 1), lambda i: (i, 0)),
        ],
        out_specs=pl.BlockSpec((BR, D), lambda i: (i, 0)),
        out_shape=jax.ShapeDtypeStruct((N, D), jnp.float32),
    )(x, W1, deg_col)


def _mid_body(agg_ref, h_ref, deg_ref, b_ref, w_ref, o_ref):
    dinv = lax.rsqrt(deg_ref[...])
    t = (agg_ref[0] + agg_ref[1] + h_ref[...]) * dinv + b_ref[...]
    r = jnp.maximum(t, 0.0)
    o_ref[...] = jnp.dot(r, w_ref[...],
                         preferred_element_type=jnp.float32) * dinv


def _mid(agg, h1p, deg_col, b1, W2):
    return pl.pallas_call(
        _mid_body,
        grid=(N // BR,),
        in_specs=[
            pl.BlockSpec((NC, BR, D), lambda i: (0, i, 0)),
            pl.BlockSpec((BR, D), lambda i: (i, 0)),
            pl.BlockSpec((BR, 1), lambda i: (i, 0)),
            pl.BlockSpec((1, D), lambda i: (0, 0)),
            pl.BlockSpec((D, D), lambda i: (0, 0)),
        ],
        out_specs=pl.BlockSpec((BR, D), lambda i: (i, 0)),
        out_shape=jax.ShapeDtypeStruct((N, D), jnp.float32),
    )(agg, h1p, deg_col, b1, W2)


def _final_body(agg_ref, h_ref, deg_ref, b_ref, batch_ref, wout_ref, bout_ref,
                o_ref, pooled):
    i = pl.program_id(0)
    dinv = lax.rsqrt(deg_ref[...])
    t = (agg_ref[0] + agg_ref[1] + h_ref[...]) * dinv + b_ref[...]
    gids = lax.broadcasted_iota(jnp.int32, (G, BR), 0)
    onehot = jnp.where(batch_ref[0] == gids, 1.0, 0.0)
    part = jnp.dot(onehot, t, preferred_element_type=jnp.float32)

    @pl.when(i == 0)
    def _init():
        pooled[...] = part

    @pl.when(i > 0)
    def _acc():
        pooled[...] += part

    @pl.when(i == pl.num_programs(0) - 1)
    def _fin():
        logits = jnp.dot(pooled[...], wout_ref[...],
                         preferred_element_type=jnp.float32) + bout_ref[...]
        m = jnp.max(logits, axis=-1, keepdims=True)
        s = logits - m
        lse = jnp.log(jnp.sum(jnp.exp(s), axis=-1, keepdims=True))
        o_ref[...] = s - lse


def _final(agg, h2p, deg_col, b2, batch2d, W_out, b_out):
    return pl.pallas_call(
        _final_body,
        grid=(N // BR,),
        in_specs=[
            pl.BlockSpec((NC, BR, D), lambda i: (0, i, 0)),
            pl.BlockSpec((BR, D), lambda i: (i, 0)),
            pl.BlockSpec((BR, 1), lambda i: (i, 0)),
            pl.BlockSpec((1, D), lambda i: (0, 0)),
            pl.BlockSpec((1, 1, BR), lambda i: (i, 0, 0)),
            pl.BlockSpec((D, C), lambda i: (0, 0)),
            pl.BlockSpec((1, C), lambda i: (0, 0)),
        ],
        out_specs=pl.BlockSpec((G, C), lambda i: (0, 0)),
        out_shape=jax.ShapeDtypeStruct((G, C), jnp.float32),
        scratch_shapes=[pltpu.VMEM((G, D), jnp.float32)],
    )(agg, h2p, deg_col, b2, batch2d, W_out, b_out)


def kernel(x, edge_index, batch, W1, b1, W2, b2, W_out, b_out):
    e = edge_index.shape[1]
    cpt = -(-e // (NW * CHUNK))
    cpt += (-cpt) % (NBUF * NPHASE * 4)
    main_e = (NW - 1) * cpt * CHUNK
    assert e % CHUNK == 0 and main_e <= e <= NW * cpt * CHUNK
    ei3 = edge_index.reshape(2, e // CHUNK, CHUNK)
    tail = edge_index[:, main_e:]
    padlen = NW * cpt * CHUNK - e
    ar = jnp.arange(padlen, dtype=jnp.int32)
    pad3 = jnp.concatenate(
        [tail, jnp.stack([ar % N, N + ar % (N_ACC - N)])], axis=1,
    ).reshape(2, cpt, CHUNK)

    deg2 = _make_deg(cpt)(ei3, pad3)
    deg_col = (deg2[0, 0, :N] + deg2[1, 0, :N] + 1.0).reshape(N, 1)

    h1p = _mm1(x, W1, deg_col)
    agg1 = _make_agg(cpt)(h1p, ei3, pad3)
    h2p = _mid(agg1, h1p, deg_col, b1.reshape(1, D), W2)
    agg2 = _make_agg(cpt)(h2p, ei3, pad3)
    batch3d = batch.reshape(N // BR, 1, BR)
    return _final(agg2, h2p, deg_col, b2.reshape(1, D), batch3d,
                  W_out, b_out.reshape(1, C))
